# baseline (speedup 1.0000x reference)
import functools

import jax
import jax.numpy as jnp
from jax import lax
from jax.experimental import pallas as pl
from jax.experimental.pallas import tpu as pltpu
from jax.experimental.pallas import tpu_sc as plsc

MAXLEN = 200
EMBED_DIM = 128
ROW = MAXLEN * EMBED_DIM
NC = 2
NS = 16
NW = NC * NS
LANES = 16
UNROLL = 16
VECS_PER_ROW = ROW // LANES
B_BLK = 128
SC_ROWS = 1024


def _tc_add_kernel(x_ref, pos_ref, out_ref):
    out_ref[...] = x_ref[...] + pos_ref[...][None, :, :]


def _tc_part(x, pos_table, tc_rows):
    return pl.pallas_call(
        _tc_add_kernel,
        grid=(tc_rows // B_BLK,),
        in_specs=[
            pl.BlockSpec((B_BLK, MAXLEN, EMBED_DIM), lambda i: (i, 0, 0)),
            pl.BlockSpec((MAXLEN, EMBED_DIM), lambda i: (0, 0)),
        ],
        out_specs=pl.BlockSpec((B_BLK, MAXLEN, EMBED_DIM), lambda i: (i, 0, 0)),
        out_shape=jax.ShapeDtypeStruct((tc_rows, MAXLEN, EMBED_DIM), x.dtype),
    )(x, pos_table)


def _make_sc_kernel(batch, row0, sc_rows):
    n = sc_rows // NW
    mesh = plsc.VectorSubcoreMesh(core_axis_name="c", subcore_axis_name="s")

    @functools.partial(
        pl.kernel,
        mesh=mesh,
        out_type=jax.ShapeDtypeStruct((sc_rows, ROW), jnp.float32),
        scratch_types=[
            pltpu.VMEM((ROW,), jnp.float32),
            pltpu.VMEM((ROW,), jnp.float32),
            pltpu.VMEM((ROW,), jnp.float32),
            pltpu.VMEM((ROW,), jnp.float32),
            pltpu.VMEM((ROW,), jnp.float32),
            pltpu.SemaphoreType.DMA,
            pltpu.SemaphoreType.DMA,
            pltpu.SemaphoreType.DMA,
            pltpu.SemaphoreType.DMA,
        ],
    )
    def sc_add(x_hbm, pos_hbm, out_hbm, pos_v, xb0, xb1, ob0, ob1,
               sin0, sin1, sout0, sout1):
        wid = lax.axis_index("s") * NC + lax.axis_index("c")
        base = wid * n
        pltpu.sync_copy(pos_hbm, pos_v)

        def in_copy(row, buf, sem):
            return pltpu.make_async_copy(x_hbm.at[row0 + row], buf, sem)

        def out_copy(buf, row, sem):
            return pltpu.make_async_copy(buf, out_hbm.at[row], sem)

        def compute(src, dst):
            @plsc.parallel_loop(0, VECS_PER_ROW, step=1, unroll=UNROLL)
            def body(i):
                off = i * LANES
                dst[pl.ds(off, LANES)] = (
                    src[pl.ds(off, LANES)] + pos_v[pl.ds(off, LANES)]
                )

        in_copy(base + 0, xb0, sin0).start()
        in_copy(base + 1, xb1, sin1).start()

        in_copy(base + 0, xb0, sin0).wait()
        compute(xb0, ob0)
        out_copy(ob0, base + 0, sout0).start()
        in_copy(base + 2, xb0, sin0).start()

        in_copy(base + 1, xb1, sin1).wait()
        compute(xb1, ob1)
        out_copy(ob1, base + 1, sout1).start()
        in_copy(base + 3, xb1, sin1).start()

        def main_body(k, c):
            re = base + 2 + 2 * k
            in_copy(re, xb0, sin0).wait()
            out_copy(ob0, re, sout0).wait()
            compute(xb0, ob0)
            out_copy(ob0, re, sout0).start()
            in_copy(re + 2, xb0, sin0).start()

            ro = re + 1
            in_copy(ro, xb1, sin1).wait()
            out_copy(ob1, ro, sout1).wait()
            compute(xb1, ob1)
            out_copy(ob1, ro, sout1).start()
            in_copy(ro + 2, xb1, sin1).start()
            return c

        lax.fori_loop(0, (n - 4) // 2, main_body, 0)

        re = base + n - 2
        in_copy(re, xb0, sin0).wait()
        out_copy(ob0, re, sout0).wait()
        compute(xb0, ob0)
        out_copy(ob0, re, sout0).start()

        ro = base + n - 1
        in_copy(ro, xb1, sin1).wait()
        out_copy(ob1, ro, sout1).wait()
        compute(xb1, ob1)
        out_copy(ob1, ro, sout1).start()

        out_copy(ob0, re, sout0).wait()
        out_copy(ob1, ro, sout1).wait()

    return sc_add


def kernel(x, pos_table):
    batch = x.shape[0]
    tc_rows = batch - SC_ROWS
    x2 = x.reshape(batch, ROW)
    pos2 = pos_table.reshape(ROW)
    out_sc = _make_sc_kernel(batch, tc_rows, SC_ROWS)(x2, pos2)
    out_tc = _tc_part(x, pos_table, tc_rows)
    out = jnp.concatenate(
        [out_tc, out_sc.reshape(SC_ROWS, MAXLEN, EMBED_DIM)], axis=0)
    return out


# speedup vs baseline: 2.8497x; 2.8497x over previous
import functools

import jax
import jax.numpy as jnp
from jax import lax
from jax.experimental import pallas as pl
from jax.experimental.pallas import tpu as pltpu
from jax.experimental.pallas import tpu_sc as plsc

MAXLEN = 200
EMBED_DIM = 128
NC = 2
NS = 16
NW = NC * NS
LANES = 16
COLS = EMBED_DIM // LANES


def _make_sc_kernel(batch):
    n = batch // NW
    mesh = plsc.VectorSubcoreMesh(core_axis_name="c", subcore_axis_name="s")

    @functools.partial(
        pl.kernel,
        mesh=mesh,
        out_type=jax.ShapeDtypeStruct((batch, MAXLEN, EMBED_DIM), jnp.float32),
        compiler_params=pltpu.CompilerParams(use_tc_tiling_on_sc=True),
        scratch_types=[
            pltpu.VMEM((MAXLEN, EMBED_DIM), jnp.float32),
            pltpu.VMEM((MAXLEN, EMBED_DIM), jnp.float32),
            pltpu.VMEM((MAXLEN, EMBED_DIM), jnp.float32),
            pltpu.VMEM((MAXLEN, EMBED_DIM), jnp.float32),
            pltpu.VMEM((MAXLEN, EMBED_DIM), jnp.float32),
            pltpu.SemaphoreType.DMA,
            pltpu.SemaphoreType.DMA,
            pltpu.SemaphoreType.DMA,
            pltpu.SemaphoreType.DMA,
        ],
    )
    def sc_add(x_hbm, pos_hbm, out_hbm, pos_v, xb0, xb1, ob0, ob1,
               sin0, sin1, sout0, sout1):
        wid = lax.axis_index("s") * NC + lax.axis_index("c")
        base = wid * n
        pltpu.sync_copy(pos_hbm, pos_v)

        def in_copy(row, buf, sem):
            return pltpu.make_async_copy(x_hbm.at[row], buf, sem)

        def out_copy(buf, row, sem):
            return pltpu.make_async_copy(buf, out_hbm.at[row], sem)

        def compute(src, dst):
            @plsc.parallel_loop(0, MAXLEN, step=1, unroll=4)
            def body(r):
                for u in range(COLS):
                    dst[r, pl.ds(u * LANES, LANES)] = (
                        src[r, pl.ds(u * LANES, LANES)]
                        + pos_v[r, pl.ds(u * LANES, LANES)]
                    )

        in_copy(base + 0, xb0, sin0).start()
        in_copy(base + 1, xb1, sin1).start()

        in_copy(base + 0, xb0, sin0).wait()
        compute(xb0, ob0)
        out_copy(ob0, base + 0, sout0).start()
        in_copy(base + 2, xb0, sin0).start()

        in_copy(base + 1, xb1, sin1).wait()
        compute(xb1, ob1)
        out_copy(ob1, base + 1, sout1).start()
        in_copy(base + 3, xb1, sin1).start()

        def main_body(k, c):
            re = base + 2 + 2 * k
            in_copy(re, xb0, sin0).wait()
            out_copy(ob0, re, sout0).wait()
            compute(xb0, ob0)
            out_copy(ob0, re, sout0).start()
            in_copy(re + 2, xb0, sin0).start()

            ro = re + 1
            in_copy(ro, xb1, sin1).wait()
            out_copy(ob1, ro, sout1).wait()
            compute(xb1, ob1)
            out_copy(ob1, ro, sout1).start()
            in_copy(ro + 2, xb1, sin1).start()
            return c

        lax.fori_loop(0, (n - 4) // 2, main_body, 0)

        re = base + n - 2
        in_copy(re, xb0, sin0).wait()
        out_copy(ob0, re, sout0).wait()
        compute(xb0, ob0)
        out_copy(ob0, re, sout0).start()

        ro = base + n - 1
        in_copy(ro, xb1, sin1).wait()
        out_copy(ob1, ro, sout1).wait()
        compute(xb1, ob1)
        out_copy(ob1, ro, sout1).start()

        out_copy(ob0, re, sout0).wait()
        out_copy(ob1, ro, sout1).wait()

    return sc_add


def kernel(x, pos_table):
    return _make_sc_kernel(x.shape[0])(x, pos_table)
